# baseline (speedup 1.0000x reference)
import jax
import jax.numpy as jnp
from jax import lax
from jax.experimental import pallas as pl
from jax.experimental.pallas import tpu as pltpu
from jax.experimental.pallas import tpu_sc as plsc

_B = 16384
_M = 14
_D = 64
_NC = 2
_NS = 16
_L = 16
_NW = _NC * _NS
_PER_W = _B // _NW
_C = 16
_NCHUNK = _PER_W // _C
_NG = _C // _L
_NR = 1400000
_NT = (_NR + 127) // 128
_K = 32
_NTP = ((_NT + _K - 1) // _K) * _K


def _copy_body(src, dst):
    for t in range(_K):
        dst[0, t] = src[:, t * 128:(t + 1) * 128]


def _launder(table_t):
    grid = (8, (_NT + _K - 1) // _K)
    out = pl.pallas_call(
        _copy_body,
        grid=grid,
        in_specs=[pl.BlockSpec((8, 128 * _K), lambda a, u: (a, u))],
        out_specs=pl.BlockSpec((1, _K, 8, 128), lambda a, u: (a, u, 0, 0)),
        out_shape=jax.ShapeDtypeStruct((8, grid[1] * _K, 8, 128),
                                       jnp.float32),
    )(table_t)
    return out.reshape(-1, 16)


def _impact_body(uids, iids, users, items, nbs, out,
                 uidx_all, iidx_all, nb_all, out_all, eidx,
                 u0, u1, e0, e1, sem_nb, sem0, sem1):
    wid = lax.axis_index("s") * _NC + lax.axis_index("c")
    base0 = wid * _PER_W
    iota = lax.iota(jnp.int32, _L)
    ubufs = (u0, u1)
    ebufs = (e0, e1)
    sems = (sem0, sem1)
    _EC = 2 * _D * _C
    _SLOT = _EC + _C

    pltpu.sync_copy(uids.at[pl.ds(base0, _PER_W)], uidx_all)
    pltpu.sync_copy(iids.at[pl.ds(base0, _PER_W)], iidx_all)
    nbcp = pltpu.async_copy(nbs.at[iidx_all], nb_all, sem_nb)

    def issue(n, s):
        rlo = iidx_all[pl.ds(n * _C, _L)] * _M + 1
        t1 = rlo >> 7
        lane = rlo & 127
        straddle = lane >= 116
        tp1 = (t1 << 10) + lane
        tp2 = (t1 + 1) << 10
        for c in range(_D):
            base_c = ((c >> 3) * _NTP << 10) + ((c & 7) << 7)
            a_v = (tp1 + base_c) >> 4
            b_v = (tp2 + base_c) >> 4
            part = jnp.where(straddle, b_v, a_v + 1)
            eidx[pl.ds(s * _SLOT + c * 2 * _C, _L)] = a_v
            eidx[pl.ds(s * _SLOT + c * 2 * _C + _L, _L)] = part
        eidx[pl.ds(s * _SLOT + _EC, _L)] = uidx_all[pl.ds(n * _C, _L)]
        pltpu.async_copy(items.at[eidx.at[pl.ds(s * _SLOT, _EC)]],
                         ebufs[s], sems[s])
        pltpu.async_copy(users.at[eidx.at[pl.ds(s * _SLOT + _EC, _C)]],
                         ubufs[s], sems[s])

    def drain(s):
        pltpu.make_async_copy(items.at[eidx.at[pl.ds(0, _EC)]],
                              ebufs[s], sems[s]).wait()
        pltpu.make_async_copy(users.at[eidx.at[pl.ds(0, _C)]],
                              ubufs[s], sems[s]).wait()

    def compute(n, s):
        urows_v = ubufs[s]
        erows_v = ebufs[s]
        rows = iota
        nb = nb_all[pl.ds(n * _C, _L)]
        rlo = iidx_all[pl.ds(n * _C, _L)] * _M + 1
        t1 = rlo >> 7
        plo15 = rlo & 15
        qh = []
        lo = []
        for j in range(1, _M):
            rj = rlo + (j - 1)
            seg2 = (rj >> 7) > t1
            pos = jnp.where(seg2, 16 + (rj & 127), plo15 + (j - 1))
            qh.append(rows + ((pos >> 4) << 4))
            lo.append(pos & 15)

        def cstep(c, accs):
            csplat = jnp.full((_L,), 0, jnp.int32) + c
            u_c = plsc.load_gather(urows_v, [rows, csplat])
            cc = c * 2 * _C
            new = []
            for j in range(1, _M):
                e = plsc.load_gather(erows_v, [qh[j - 1] + cc, lo[j - 1]])
                dv = u_c - e
                new.append(accs[j - 1] + dv * dv)
            return tuple(new)

        accs = lax.fori_loop(
            0, _D, cstep,
            tuple(jnp.zeros((_L,), jnp.float32) for _ in range(_M - 1)))

        best = accs[0]
        bidx = jnp.full((_L,), 1.0, jnp.float32)
        for j in range(2, _M):
            upd = (nb >= j) & (accs[j - 1] < best)
            best = jnp.where(upd, accs[j - 1], best)
            bidx = jnp.where(upd, jnp.float32(j), bidx)
        nbf = nb.astype(jnp.float32)
        out_all[pl.ds(n * _C, _L)] = (bidx - 1.0) / (nbf - 1.0) + 1.0

    nbcp.wait()
    issue(0, 0)
    issue(1, 1)

    def step(k, carry):
        n0 = 2 * k
        drain(0)
        compute(n0, 0)
        issue(n0 + 2, 0)
        drain(1)
        compute(n0 + 1, 1)
        issue(n0 + 3, 1)
        return carry

    lax.fori_loop(0, (_NCHUNK - 2) // 2, step, 0)
    drain(0)
    compute(_NCHUNK - 2, 0)
    drain(1)
    compute(_NCHUNK - 1, 1)
    pltpu.sync_copy(out_all, out.at[pl.ds(base0, _PER_W)])


@jax.jit
def kernel(user_ids, item_ids, concept_ids, users_w, item_resp_w,
           nb_modalities, mask):
    del concept_ids, mask
    items_runs = _launder(jnp.swapaxes(item_resp_w, 0, 1))
    run = pl.kernel(
        _impact_body,
        out_type=jax.ShapeDtypeStruct((_B,), jnp.float32),
        mesh=plsc.VectorSubcoreMesh(core_axis_name="c", subcore_axis_name="s",
                                    num_cores=_NC, num_subcores=_NS),
        compiler_params=pltpu.CompilerParams(needs_layout_passes=False,
                                             use_tc_tiling_on_sc=False),
        scratch_types=[
            pltpu.VMEM((_PER_W,), jnp.int32),
            pltpu.VMEM((_PER_W,), jnp.int32),
            pltpu.VMEM((_PER_W,), jnp.int32),
            pltpu.VMEM((_PER_W,), jnp.float32),
            pltpu.VMEM((2 * (2 * _D * _C + _C),), jnp.int32),
            pltpu.VMEM((_C, _D), jnp.float32),
            pltpu.VMEM((_C, _D), jnp.float32),
            pltpu.VMEM((2 * _D * _C, 16), jnp.float32),
            pltpu.VMEM((2 * _D * _C, 16), jnp.float32),
            pltpu.SemaphoreType.DMA,
            pltpu.SemaphoreType.DMA,
            pltpu.SemaphoreType.DMA,
        ],
    )
    return run(user_ids.astype(jnp.int32), item_ids.astype(jnp.int32),
               users_w, items_runs, nb_modalities.astype(jnp.int32))


# speedup vs baseline: 2.5294x; 2.5294x over previous
import jax
import jax.numpy as jnp
from jax import lax
from jax.experimental import pallas as pl
from jax.experimental.pallas import tpu as pltpu
from jax.experimental.pallas import tpu_sc as plsc

_B = 16384
_M = 14
_D = 64
_NC = 2
_NS = 16
_L = 16
_NW = _NC * _NS
_PER_W = _B // _NW
_C = 16
_NCHUNK = _PER_W // _C
_NG = _C // _L
_NR = 1400000
_NT = (_NR + 127) // 128
_K = 128
_NTP = ((_NT + _K - 1) // _K) * _K


def _copy_body(src, dst):
    for t in range(_K):
        dst[0, t] = src[:, t * 128:(t + 1) * 128]


def _launder(table_t):
    grid = (8, (_NT + _K - 1) // _K)
    out = pl.pallas_call(
        _copy_body,
        grid=grid,
        in_specs=[pl.BlockSpec((8, 128 * _K), lambda a, u: (a, u))],
        out_specs=pl.BlockSpec((1, _K, 8, 128), lambda a, u: (a, u, 0, 0)),
        out_shape=jax.ShapeDtypeStruct((8, grid[1] * _K, 8, 128),
                                       jnp.float32),
    )(table_t)
    return out.reshape(-1, 16)


def _impact_body(uids, iids, users, items, nbs, out,
                 uidx_all, iidx_all, nb_all, out_all, eidx,
                 u0, u1, e0, e1, sem_nb, sem0, sem1):
    wid = lax.axis_index("s") * _NC + lax.axis_index("c")
    base0 = wid * _PER_W
    iota = lax.iota(jnp.int32, _L)
    ubufs = (u0, u1)
    ebufs = (e0, e1)
    sems = (sem0, sem1)
    _EC = 2 * _D * _C
    _SLOT = _EC + _C

    pltpu.sync_copy(uids.at[pl.ds(base0, _PER_W)], uidx_all)
    pltpu.sync_copy(iids.at[pl.ds(base0, _PER_W)], iidx_all)
    nbcp = pltpu.async_copy(nbs.at[iidx_all], nb_all, sem_nb)

    def issue(n, s):
        rlo = iidx_all[pl.ds(n * _C, _L)] * _M + 1
        t1 = rlo >> 7
        lane = rlo & 127
        straddle = lane >= 116
        tp1 = (t1 << 10) + lane
        tp2 = (t1 + 1) << 10
        for c in range(_D):
            base_c = ((c >> 3) * _NTP << 10) + ((c & 7) << 7)
            a_v = (tp1 + base_c) >> 4
            b_v = (tp2 + base_c) >> 4
            part = jnp.where(straddle, b_v, a_v + 1)
            eidx[pl.ds(s * _SLOT + c * 2 * _C, _L)] = a_v
            eidx[pl.ds(s * _SLOT + c * 2 * _C + _L, _L)] = part
        eidx[pl.ds(s * _SLOT + _EC, _L)] = uidx_all[pl.ds(n * _C, _L)]
        pltpu.async_copy(items.at[eidx.at[pl.ds(s * _SLOT, _EC)]],
                         ebufs[s], sems[s])
        pltpu.async_copy(users.at[eidx.at[pl.ds(s * _SLOT + _EC, _C)]],
                         ubufs[s], sems[s])

    def drain(s):
        pltpu.make_async_copy(items.at[eidx.at[pl.ds(0, _EC)]],
                              ebufs[s], sems[s]).wait()
        pltpu.make_async_copy(users.at[eidx.at[pl.ds(0, _C)]],
                              ubufs[s], sems[s]).wait()

    def compute(n, s):
        urows_v = ubufs[s]
        erows_v = ebufs[s]
        rows = iota
        nb = nb_all[pl.ds(n * _C, _L)]
        rlo = iidx_all[pl.ds(n * _C, _L)] * _M + 1
        t1 = rlo >> 7
        plo15 = rlo & 15
        qh = []
        lo = []
        for j in range(1, _M):
            rj = rlo + (j - 1)
            seg2 = (rj >> 7) > t1
            pos = jnp.where(seg2, 16 + (rj & 127), plo15 + (j - 1))
            qh.append(rows + ((pos >> 4) << 4))
            lo.append(pos & 15)

        def cstep(c, accs):
            csplat = jnp.full((_L,), 0, jnp.int32) + c
            u_c = plsc.load_gather(urows_v, [rows, csplat])
            cc = c * 2 * _C
            new = []
            for j in range(1, _M):
                e = plsc.load_gather(erows_v, [qh[j - 1] + cc, lo[j - 1]])
                dv = u_c - e
                new.append(accs[j - 1] + dv * dv)
            return tuple(new)

        accs = lax.fori_loop(
            0, _D, cstep,
            tuple(jnp.zeros((_L,), jnp.float32) for _ in range(_M - 1)))

        best = accs[0]
        bidx = jnp.full((_L,), 1.0, jnp.float32)
        for j in range(2, _M):
            upd = (nb >= j) & (accs[j - 1] < best)
            best = jnp.where(upd, accs[j - 1], best)
            bidx = jnp.where(upd, jnp.float32(j), bidx)
        nbf = nb.astype(jnp.float32)
        out_all[pl.ds(n * _C, _L)] = (bidx - 1.0) / (nbf - 1.0) + 1.0

    nbcp.wait()
    issue(0, 0)
    issue(1, 1)

    def step(k, carry):
        n0 = 2 * k
        drain(0)
        compute(n0, 0)
        issue(n0 + 2, 0)
        drain(1)
        compute(n0 + 1, 1)
        issue(n0 + 3, 1)
        return carry

    lax.fori_loop(0, (_NCHUNK - 2) // 2, step, 0)
    drain(0)
    compute(_NCHUNK - 2, 0)
    drain(1)
    compute(_NCHUNK - 1, 1)
    pltpu.sync_copy(out_all, out.at[pl.ds(base0, _PER_W)])


@jax.jit
def kernel(user_ids, item_ids, concept_ids, users_w, item_resp_w,
           nb_modalities, mask):
    del concept_ids, mask
    items_runs = _launder(jnp.swapaxes(item_resp_w, 0, 1))
    run = pl.kernel(
        _impact_body,
        out_type=jax.ShapeDtypeStruct((_B,), jnp.float32),
        mesh=plsc.VectorSubcoreMesh(core_axis_name="c", subcore_axis_name="s",
                                    num_cores=_NC, num_subcores=_NS),
        compiler_params=pltpu.CompilerParams(needs_layout_passes=False,
                                             use_tc_tiling_on_sc=False),
        scratch_types=[
            pltpu.VMEM((_PER_W,), jnp.int32),
            pltpu.VMEM((_PER_W,), jnp.int32),
            pltpu.VMEM((_PER_W,), jnp.int32),
            pltpu.VMEM((_PER_W,), jnp.float32),
            pltpu.VMEM((2 * (2 * _D * _C + _C),), jnp.int32),
            pltpu.VMEM((_C, _D), jnp.float32),
            pltpu.VMEM((_C, _D), jnp.float32),
            pltpu.VMEM((2 * _D * _C, 16), jnp.float32),
            pltpu.VMEM((2 * _D * _C, 16), jnp.float32),
            pltpu.SemaphoreType.DMA,
            pltpu.SemaphoreType.DMA,
            pltpu.SemaphoreType.DMA,
        ],
    )
    return run(user_ids.astype(jnp.int32), item_ids.astype(jnp.int32),
               users_w, items_runs, nb_modalities.astype(jnp.int32))


# speedup vs baseline: 3.6894x; 1.4586x over previous
import jax
import jax.numpy as jnp
from jax import lax
from jax.experimental import pallas as pl
from jax.experimental.pallas import tpu as pltpu
from jax.experimental.pallas import tpu_sc as plsc

_B = 16384
_M = 14
_D = 64
_NC = 2
_NS = 16
_L = 16
_NW = _NC * _NS
_PER_W = _B // _NW
_C = 16
_NCHUNK = _PER_W // _C
_NG = _C // _L
_NR = 1400000
_NT = (_NR + 127) // 128
_K = 342
_NTP = ((_NT + _K - 1) // _K) * _K


def _copy_body(src, dst):
    for t in range(_K):
        dst[0, t] = src[:, t * 128:(t + 1) * 128]


def _launder(table_t):
    grid = (8, (_NT + _K - 1) // _K)
    out = pl.pallas_call(
        _copy_body,
        grid=grid,
        in_specs=[pl.BlockSpec((8, 128 * _K), lambda a, u: (a, u))],
        out_specs=pl.BlockSpec((1, _K, 8, 128), lambda a, u: (a, u, 0, 0)),
        out_shape=jax.ShapeDtypeStruct((8, grid[1] * _K, 8, 128),
                                       jnp.float32),
    )(table_t)
    return out.reshape(-1, 16)


def _impact_body(uids, iids, users, items, nbs, out,
                 uidx_all, iidx_all, nb_all, out_all, eidx,
                 u0, u1, e0, e1, sem_nb, sem0, sem1):
    wid = lax.axis_index("s") * _NC + lax.axis_index("c")
    base0 = wid * _PER_W
    iota = lax.iota(jnp.int32, _L)
    ubufs = (u0, u1)
    ebufs = (e0, e1)
    sems = (sem0, sem1)
    _EC = 2 * _D * _C
    _SLOT = _EC + _C

    pltpu.sync_copy(uids.at[pl.ds(base0, _PER_W)], uidx_all)
    pltpu.sync_copy(iids.at[pl.ds(base0, _PER_W)], iidx_all)
    nbcp = pltpu.async_copy(nbs.at[iidx_all], nb_all, sem_nb)

    def issue(n, s):
        rlo = iidx_all[pl.ds(n * _C, _L)] * _M + 1
        t1 = rlo >> 7
        lane = rlo & 127
        straddle = lane >= 116
        tp1 = (t1 << 10) + lane
        tp2 = (t1 + 1) << 10
        for c in range(_D):
            base_c = ((c >> 3) * _NTP << 10) + ((c & 7) << 7)
            a_v = (tp1 + base_c) >> 4
            b_v = (tp2 + base_c) >> 4
            part = jnp.where(straddle, b_v, a_v + 1)
            eidx[pl.ds(s * _SLOT + c * 2 * _C, _L)] = a_v
            eidx[pl.ds(s * _SLOT + c * 2 * _C + _L, _L)] = part
        eidx[pl.ds(s * _SLOT + _EC, _L)] = uidx_all[pl.ds(n * _C, _L)]
        pltpu.async_copy(items.at[eidx.at[pl.ds(s * _SLOT, _EC)]],
                         ebufs[s], sems[s])
        pltpu.async_copy(users.at[eidx.at[pl.ds(s * _SLOT + _EC, _C)]],
                         ubufs[s], sems[s])

    def drain(s):
        pltpu.make_async_copy(items.at[eidx.at[pl.ds(0, _EC)]],
                              ebufs[s], sems[s]).wait()
        pltpu.make_async_copy(users.at[eidx.at[pl.ds(0, _C)]],
                              ubufs[s], sems[s]).wait()

    def compute(n, s):
        urows_v = ubufs[s]
        erows_v = ebufs[s]
        rows = iota
        nb = nb_all[pl.ds(n * _C, _L)]
        rlo = iidx_all[pl.ds(n * _C, _L)] * _M + 1
        t1 = rlo >> 7
        plo15 = rlo & 15
        qh = []
        lo = []
        for j in range(1, _M):
            rj = rlo + (j - 1)
            seg2 = (rj >> 7) > t1
            pos = jnp.where(seg2, 16 + (rj & 127), plo15 + (j - 1))
            qh.append(rows + ((pos >> 4) << 4))
            lo.append(pos & 15)

        def cstep(c, accs):
            csplat = jnp.full((_L,), 0, jnp.int32) + c
            u_c = plsc.load_gather(urows_v, [rows, csplat])
            cc = c * 2 * _C
            new = []
            for j in range(1, _M):
                e = plsc.load_gather(erows_v, [qh[j - 1] + cc, lo[j - 1]])
                dv = u_c - e
                new.append(accs[j - 1] + dv * dv)
            return tuple(new)

        accs = lax.fori_loop(
            0, _D, cstep,
            tuple(jnp.zeros((_L,), jnp.float32) for _ in range(_M - 1)))

        best = accs[0]
        bidx = jnp.full((_L,), 1.0, jnp.float32)
        for j in range(2, _M):
            upd = (nb >= j) & (accs[j - 1] < best)
            best = jnp.where(upd, accs[j - 1], best)
            bidx = jnp.where(upd, jnp.float32(j), bidx)
        nbf = nb.astype(jnp.float32)
        out_all[pl.ds(n * _C, _L)] = (bidx - 1.0) / (nbf - 1.0) + 1.0

    nbcp.wait()
    issue(0, 0)
    issue(1, 1)

    def step(k, carry):
        n0 = 2 * k
        drain(0)
        compute(n0, 0)
        issue(n0 + 2, 0)
        drain(1)
        compute(n0 + 1, 1)
        issue(n0 + 3, 1)
        return carry

    lax.fori_loop(0, (_NCHUNK - 2) // 2, step, 0)
    drain(0)
    compute(_NCHUNK - 2, 0)
    drain(1)
    compute(_NCHUNK - 1, 1)
    pltpu.sync_copy(out_all, out.at[pl.ds(base0, _PER_W)])


@jax.jit
def kernel(user_ids, item_ids, concept_ids, users_w, item_resp_w,
           nb_modalities, mask):
    del concept_ids, mask
    items_runs = _launder(jnp.swapaxes(item_resp_w, 0, 1))
    run = pl.kernel(
        _impact_body,
        out_type=jax.ShapeDtypeStruct((_B,), jnp.float32),
        mesh=plsc.VectorSubcoreMesh(core_axis_name="c", subcore_axis_name="s",
                                    num_cores=_NC, num_subcores=_NS),
        compiler_params=pltpu.CompilerParams(needs_layout_passes=False,
                                             use_tc_tiling_on_sc=False),
        scratch_types=[
            pltpu.VMEM((_PER_W,), jnp.int32),
            pltpu.VMEM((_PER_W,), jnp.int32),
            pltpu.VMEM((_PER_W,), jnp.int32),
            pltpu.VMEM((_PER_W,), jnp.float32),
            pltpu.VMEM((2 * (2 * _D * _C + _C),), jnp.int32),
            pltpu.VMEM((_C, _D), jnp.float32),
            pltpu.VMEM((_C, _D), jnp.float32),
            pltpu.VMEM((2 * _D * _C, 16), jnp.float32),
            pltpu.VMEM((2 * _D * _C, 16), jnp.float32),
            pltpu.SemaphoreType.DMA,
            pltpu.SemaphoreType.DMA,
            pltpu.SemaphoreType.DMA,
        ],
    )
    return run(user_ids.astype(jnp.int32), item_ids.astype(jnp.int32),
               users_w, items_runs, nb_modalities.astype(jnp.int32))


# speedup vs baseline: 4.1918x; 1.1362x over previous
import jax
import jax.numpy as jnp
from jax import lax
from jax.experimental import pallas as pl
from jax.experimental.pallas import tpu as pltpu
from jax.experimental.pallas import tpu_sc as plsc

_B = 16384
_M = 14
_D = 64
_NC = 2
_NS = 16
_L = 16
_NW = _NC * _NS
_PER_W = _B // _NW
_C = 16
_NCHUNK = _PER_W // _C
_NG = _C // _L
_NR = 1400000
_NT = (_NR + 127) // 128
_K = 684
_NTP = ((_NT + _K - 1) // _K) * _K


def _copy_body(src, dst):
    for t in range(_K):
        dst[0, t] = src[:, t * 128:(t + 1) * 128]


def _launder(table_t):
    grid = (8, (_NT + _K - 1) // _K)
    out = pl.pallas_call(
        _copy_body,
        grid=grid,
        in_specs=[pl.BlockSpec((8, 128 * _K), lambda a, u: (a, u))],
        out_specs=pl.BlockSpec((1, _K, 8, 128), lambda a, u: (a, u, 0, 0)),
        out_shape=jax.ShapeDtypeStruct((8, grid[1] * _K, 8, 128),
                                       jnp.float32),
    )(table_t)
    return out.reshape(-1, 16)


def _impact_body(uids, iids, users, items, nbs, out,
                 uidx_all, iidx_all, nb_all, out_all, eidx,
                 u0, u1, e0, e1, sem_nb, sem0, sem1):
    wid = lax.axis_index("s") * _NC + lax.axis_index("c")
    base0 = wid * _PER_W
    iota = lax.iota(jnp.int32, _L)
    ubufs = (u0, u1)
    ebufs = (e0, e1)
    sems = (sem0, sem1)
    _EC = 2 * _D * _C
    _SLOT = _EC + _C

    pltpu.sync_copy(uids.at[pl.ds(base0, _PER_W)], uidx_all)
    pltpu.sync_copy(iids.at[pl.ds(base0, _PER_W)], iidx_all)
    nbcp = pltpu.async_copy(nbs.at[iidx_all], nb_all, sem_nb)

    def issue(n, s):
        rlo = iidx_all[pl.ds(n * _C, _L)] * _M + 1
        t1 = rlo >> 7
        lane = rlo & 127
        straddle = lane >= 116
        tp1 = (t1 << 10) + lane
        tp2 = (t1 + 1) << 10
        for c in range(_D):
            base_c = ((c >> 3) * _NTP << 10) + ((c & 7) << 7)
            a_v = (tp1 + base_c) >> 4
            b_v = (tp2 + base_c) >> 4
            part = jnp.where(straddle, b_v, a_v + 1)
            eidx[pl.ds(s * _SLOT + c * 2 * _C, _L)] = a_v
            eidx[pl.ds(s * _SLOT + c * 2 * _C + _L, _L)] = part
        eidx[pl.ds(s * _SLOT + _EC, _L)] = uidx_all[pl.ds(n * _C, _L)]
        pltpu.async_copy(items.at[eidx.at[pl.ds(s * _SLOT, _EC)]],
                         ebufs[s], sems[s])
        pltpu.async_copy(users.at[eidx.at[pl.ds(s * _SLOT + _EC, _C)]],
                         ubufs[s], sems[s])

    def drain(s):
        pltpu.make_async_copy(items.at[eidx.at[pl.ds(0, _EC)]],
                              ebufs[s], sems[s]).wait()
        pltpu.make_async_copy(users.at[eidx.at[pl.ds(0, _C)]],
                              ubufs[s], sems[s]).wait()

    def compute(n, s):
        urows_v = ubufs[s]
        erows_v = ebufs[s]
        rows = iota
        nb = nb_all[pl.ds(n * _C, _L)]
        rlo = iidx_all[pl.ds(n * _C, _L)] * _M + 1
        t1 = rlo >> 7
        plo15 = rlo & 15
        qh = []
        lo = []
        for j in range(1, _M):
            rj = rlo + (j - 1)
            seg2 = (rj >> 7) > t1
            pos = jnp.where(seg2, 16 + (rj & 127), plo15 + (j - 1))
            qh.append(rows + ((pos >> 4) << 4))
            lo.append(pos & 15)

        def cstep(c, accs):
            csplat = jnp.full((_L,), 0, jnp.int32) + c
            u_c = plsc.load_gather(urows_v, [rows, csplat])
            cc = c * 2 * _C
            new = []
            for j in range(1, _M):
                e = plsc.load_gather(erows_v, [qh[j - 1] + cc, lo[j - 1]])
                dv = u_c - e
                new.append(accs[j - 1] + dv * dv)
            return tuple(new)

        accs = lax.fori_loop(
            0, _D, cstep,
            tuple(jnp.zeros((_L,), jnp.float32) for _ in range(_M - 1)))

        best = accs[0]
        bidx = jnp.full((_L,), 1.0, jnp.float32)
        for j in range(2, _M):
            upd = (nb >= j) & (accs[j - 1] < best)
            best = jnp.where(upd, accs[j - 1], best)
            bidx = jnp.where(upd, jnp.float32(j), bidx)
        nbf = nb.astype(jnp.float32)
        out_all[pl.ds(n * _C, _L)] = (bidx - 1.0) / (nbf - 1.0) + 1.0

    nbcp.wait()
    issue(0, 0)
    issue(1, 1)

    def step(k, carry):
        n0 = 2 * k
        drain(0)
        compute(n0, 0)
        issue(n0 + 2, 0)
        drain(1)
        compute(n0 + 1, 1)
        issue(n0 + 3, 1)
        return carry

    lax.fori_loop(0, (_NCHUNK - 2) // 2, step, 0)
    drain(0)
    compute(_NCHUNK - 2, 0)
    drain(1)
    compute(_NCHUNK - 1, 1)
    pltpu.sync_copy(out_all, out.at[pl.ds(base0, _PER_W)])


@jax.jit
def kernel(user_ids, item_ids, concept_ids, users_w, item_resp_w,
           nb_modalities, mask):
    del concept_ids, mask
    items_runs = _launder(jnp.swapaxes(item_resp_w, 0, 1))
    run = pl.kernel(
        _impact_body,
        out_type=jax.ShapeDtypeStruct((_B,), jnp.float32),
        mesh=plsc.VectorSubcoreMesh(core_axis_name="c", subcore_axis_name="s",
                                    num_cores=_NC, num_subcores=_NS),
        compiler_params=pltpu.CompilerParams(needs_layout_passes=False,
                                             use_tc_tiling_on_sc=False),
        scratch_types=[
            pltpu.VMEM((_PER_W,), jnp.int32),
            pltpu.VMEM((_PER_W,), jnp.int32),
            pltpu.VMEM((_PER_W,), jnp.int32),
            pltpu.VMEM((_PER_W,), jnp.float32),
            pltpu.VMEM((2 * (2 * _D * _C + _C),), jnp.int32),
            pltpu.VMEM((_C, _D), jnp.float32),
            pltpu.VMEM((_C, _D), jnp.float32),
            pltpu.VMEM((2 * _D * _C, 16), jnp.float32),
            pltpu.VMEM((2 * _D * _C, 16), jnp.float32),
            pltpu.SemaphoreType.DMA,
            pltpu.SemaphoreType.DMA,
            pltpu.SemaphoreType.DMA,
        ],
    )
    return run(user_ids.astype(jnp.int32), item_ids.astype(jnp.int32),
               users_w, items_runs, nb_modalities.astype(jnp.int32))


# speedup vs baseline: 4.2651x; 1.0175x over previous
import jax
import jax.numpy as jnp
from jax import lax
from jax.experimental import pallas as pl
from jax.experimental.pallas import tpu as pltpu
from jax.experimental.pallas import tpu_sc as plsc

_B = 16384
_M = 14
_D = 64
_NC = 2
_NS = 16
_L = 16
_NW = _NC * _NS
_PER_W = _B // _NW
_C = 16
_NCHUNK = _PER_W // _C
_NG = _C // _L
_NR = 1400000
_NT = (_NR + 127) // 128
_K = 1368
_NTP = ((_NT + _K - 1) // _K) * _K


def _copy_body(src, dst):
    for t in range(_K):
        dst[0, t] = src[:, t * 128:(t + 1) * 128]


def _launder(table_t):
    grid = (8, (_NT + _K - 1) // _K)
    out = pl.pallas_call(
        _copy_body,
        grid=grid,
        in_specs=[pl.BlockSpec((8, 128 * _K), lambda a, u: (a, u))],
        out_specs=pl.BlockSpec((1, _K, 8, 128), lambda a, u: (a, u, 0, 0)),
        out_shape=jax.ShapeDtypeStruct((8, grid[1] * _K, 8, 128),
                                       jnp.float32),
    )(table_t)
    return out.reshape(-1, 16)


def _impact_body(uids, iids, users, items, nbs, out,
                 uidx_all, iidx_all, nb_all, out_all, eidx,
                 u0, u1, e0, e1, sem_nb, sem0, sem1):
    wid = lax.axis_index("s") * _NC + lax.axis_index("c")
    base0 = wid * _PER_W
    iota = lax.iota(jnp.int32, _L)
    ubufs = (u0, u1)
    ebufs = (e0, e1)
    sems = (sem0, sem1)
    _EC = 2 * _D * _C
    _SLOT = _EC + _C

    pltpu.sync_copy(uids.at[pl.ds(base0, _PER_W)], uidx_all)
    pltpu.sync_copy(iids.at[pl.ds(base0, _PER_W)], iidx_all)
    nbcp = pltpu.async_copy(nbs.at[iidx_all], nb_all, sem_nb)

    def issue(n, s):
        rlo = iidx_all[pl.ds(n * _C, _L)] * _M + 1
        t1 = rlo >> 7
        lane = rlo & 127
        straddle = lane >= 116
        tp1 = (t1 << 10) + lane
        tp2 = (t1 + 1) << 10
        for c in range(_D):
            base_c = ((c >> 3) * _NTP << 10) + ((c & 7) << 7)
            a_v = (tp1 + base_c) >> 4
            b_v = (tp2 + base_c) >> 4
            part = jnp.where(straddle, b_v, a_v + 1)
            eidx[pl.ds(s * _SLOT + c * 2 * _C, _L)] = a_v
            eidx[pl.ds(s * _SLOT + c * 2 * _C + _L, _L)] = part
        eidx[pl.ds(s * _SLOT + _EC, _L)] = uidx_all[pl.ds(n * _C, _L)]
        pltpu.async_copy(items.at[eidx.at[pl.ds(s * _SLOT, _EC)]],
                         ebufs[s], sems[s])
        pltpu.async_copy(users.at[eidx.at[pl.ds(s * _SLOT + _EC, _C)]],
                         ubufs[s], sems[s])

    def drain(s):
        pltpu.make_async_copy(items.at[eidx.at[pl.ds(0, _EC)]],
                              ebufs[s], sems[s]).wait()
        pltpu.make_async_copy(users.at[eidx.at[pl.ds(0, _C)]],
                              ubufs[s], sems[s]).wait()

    def compute(n, s):
        urows_v = ubufs[s]
        erows_v = ebufs[s]
        rows = iota
        nb = nb_all[pl.ds(n * _C, _L)]
        rlo = iidx_all[pl.ds(n * _C, _L)] * _M + 1
        t1 = rlo >> 7
        plo15 = rlo & 15
        qh = []
        lo = []
        for j in range(1, _M):
            rj = rlo + (j - 1)
            seg2 = (rj >> 7) > t1
            pos = jnp.where(seg2, 16 + (rj & 127), plo15 + (j - 1))
            qh.append(rows + ((pos >> 4) << 4))
            lo.append(pos & 15)

        def cstep(c, accs):
            csplat = jnp.full((_L,), 0, jnp.int32) + c
            u_c = plsc.load_gather(urows_v, [rows, csplat])
            cc = c * 2 * _C
            new = []
            for j in range(1, _M):
                e = plsc.load_gather(erows_v, [qh[j - 1] + cc, lo[j - 1]])
                dv = u_c - e
                new.append(accs[j - 1] + dv * dv)
            return tuple(new)

        accs = lax.fori_loop(
            0, _D, cstep,
            tuple(jnp.zeros((_L,), jnp.float32) for _ in range(_M - 1)))

        best = accs[0]
        bidx = jnp.full((_L,), 1.0, jnp.float32)
        for j in range(2, _M):
            upd = (nb >= j) & (accs[j - 1] < best)
            best = jnp.where(upd, accs[j - 1], best)
            bidx = jnp.where(upd, jnp.float32(j), bidx)
        nbf = nb.astype(jnp.float32)
        out_all[pl.ds(n * _C, _L)] = (bidx - 1.0) / (nbf - 1.0) + 1.0

    nbcp.wait()
    issue(0, 0)
    issue(1, 1)

    def step(k, carry):
        n0 = 2 * k
        drain(0)
        compute(n0, 0)
        issue(n0 + 2, 0)
        drain(1)
        compute(n0 + 1, 1)
        issue(n0 + 3, 1)
        return carry

    lax.fori_loop(0, (_NCHUNK - 2) // 2, step, 0)
    drain(0)
    compute(_NCHUNK - 2, 0)
    drain(1)
    compute(_NCHUNK - 1, 1)
    pltpu.sync_copy(out_all, out.at[pl.ds(base0, _PER_W)])


@jax.jit
def kernel(user_ids, item_ids, concept_ids, users_w, item_resp_w,
           nb_modalities, mask):
    del concept_ids, mask
    items_runs = _launder(jnp.swapaxes(item_resp_w, 0, 1))
    run = pl.kernel(
        _impact_body,
        out_type=jax.ShapeDtypeStruct((_B,), jnp.float32),
        mesh=plsc.VectorSubcoreMesh(core_axis_name="c", subcore_axis_name="s",
                                    num_cores=_NC, num_subcores=_NS),
        compiler_params=pltpu.CompilerParams(needs_layout_passes=False,
                                             use_tc_tiling_on_sc=False),
        scratch_types=[
            pltpu.VMEM((_PER_W,), jnp.int32),
            pltpu.VMEM((_PER_W,), jnp.int32),
            pltpu.VMEM((_PER_W,), jnp.int32),
            pltpu.VMEM((_PER_W,), jnp.float32),
            pltpu.VMEM((2 * (2 * _D * _C + _C),), jnp.int32),
            pltpu.VMEM((_C, _D), jnp.float32),
            pltpu.VMEM((_C, _D), jnp.float32),
            pltpu.VMEM((2 * _D * _C, 16), jnp.float32),
            pltpu.VMEM((2 * _D * _C, 16), jnp.float32),
            pltpu.SemaphoreType.DMA,
            pltpu.SemaphoreType.DMA,
            pltpu.SemaphoreType.DMA,
        ],
    )
    return run(user_ids.astype(jnp.int32), item_ids.astype(jnp.int32),
               users_w, items_runs, nb_modalities.astype(jnp.int32))
